# baseline (speedup 1.0000x reference)
import functools

import jax
import jax.numpy as jnp
from jax import lax
from jax.experimental import pallas as pl
from jax.experimental.pallas import tpu as pltpu
from jax.experimental.pallas import tpu_sc as plsc

EMBED = 16
FIELD_OFFSET = 1_000_000


def _sc_gather(x_flat, table):
    info = plsc.get_sparse_core_info()
    nc, ns, lanes = info.num_cores, info.num_subcores, info.num_lanes
    nw = nc * ns
    n_idx = x_flat.shape[0]
    b_per_w = n_idx // nw
    n_sub = b_per_w // 128
    mesh = plsc.VectorSubcoreMesh(core_axis_name="c", subcore_axis_name="s")

    @functools.partial(
        pl.kernel,
        mesh=mesh,
        out_type=jax.ShapeDtypeStruct((n_idx, EMBED), jnp.float32),
        scratch_types=[
            pltpu.VMEM((b_per_w,), jnp.int32),
            pltpu.VMEM((b_per_w, EMBED), jnp.float32),
            pltpu.SemaphoreType.DMA,
        ],
        compiler_params=pltpu.CompilerParams(use_tc_tiling_on_sc=False),
    )
    def gather_k(x_hbm, table_hbm, out_hbm, idx_v, rows_v, sem):
        wid = lax.axis_index("s") * nc + lax.axis_index("c")
        base = wid * b_per_w
        pltpu.sync_copy(x_hbm.at[pl.ds(base, b_per_w)], idx_v)
        lane = lax.iota(jnp.int32, lanes)
        pat = jnp.where(lane % 2 == 1, FIELD_OFFSET, 0).astype(jnp.int32)
        for t in range(b_per_w // lanes):
            sl = pl.ds(t * lanes, lanes)
            idx_v[sl] = idx_v[sl] + pat
        copies = [
            pltpu.async_copy(
                table_hbm.at[idx_v.at[pl.ds(j * 128, 128)]],
                rows_v.at[pl.ds(j * 128, 128), :],
                sem,
            )
            for j in range(n_sub)
        ]
        for c in copies:
            c.wait()
        pltpu.sync_copy(rows_v, out_hbm.at[pl.ds(base, b_per_w)])

    return gather_k(x_flat, table)


def _tc_mlp(h, W1, b1, W2, b2, W3, b3, W4, b4):
    n_rows = h.shape[0]
    blk = 2048
    grid = (n_rows // blk,)

    def mlp_k(h_ref, w1, c1, w2, c2, w3, c3, w4, c4, o_ref):
        a = h_ref[...]
        a = jnp.maximum(
            jnp.dot(a, w1[...], preferred_element_type=jnp.float32) + c1[...], 0.0)
        a = jnp.maximum(
            jnp.dot(a, w2[...], preferred_element_type=jnp.float32) + c2[...], 0.0)
        a = jnp.maximum(
            jnp.dot(a, w3[...], preferred_element_type=jnp.float32) + c3[...], 0.0)
        a = jnp.maximum(
            jnp.dot(a, w4[...], preferred_element_type=jnp.float32) + c4[...], 0.0)
        o_ref[...] = a

    full = lambda arr: pl.BlockSpec(arr.shape, lambda i: (0, 0))
    return pl.pallas_call(
        mlp_k,
        grid=grid,
        in_specs=[
            pl.BlockSpec((blk, 32), lambda i: (i, 0)),
            full(W1), full(b1), full(W2), full(b2),
            full(W3), full(b3), full(W4), full(b4),
        ],
        out_specs=pl.BlockSpec((blk, 1), lambda i: (i, 0)),
        out_shape=jax.ShapeDtypeStruct((n_rows, 1), jnp.float32),
    )(h, W1, b1, W2, b2, W3, b3, W4, b4)


def kernel(x, table, W1, b1, W2, b2, W3, b3, W4, b4):
    n_rows = x.shape[0]
    x_flat = x.reshape(-1)
    rows = _sc_gather(x_flat, table)
    h = rows.reshape(n_rows, 2 * EMBED)
    return _tc_mlp(
        h,
        W1, b1.reshape(1, -1),
        W2, b2.reshape(1, -1),
        W3, b3.reshape(1, -1),
        W4, b4.reshape(1, -1),
    )
